# baseline (speedup 1.0000x reference)
import jax
import jax.numpy as jnp
from jax import lax
from jax.experimental import pallas as pl
from jax.experimental.pallas import tpu as pltpu
from jax.experimental.pallas import tpu_sc as plsc

VOCAB = 1000000
CTX = 200
DIM = 64
B = 1024
L = 200
LP = 208

NUM_CORES = 2
NUM_SUBCORES = 16
NW = NUM_CORES * NUM_SUBCORES
BATCHES_PER_W = B // NW
GROUPS = LP // 16


def _sc_body(xp_hbm, xh_hbm, pairs_hbm, post_hbm, out_hbm,
             idx_v, post_v, h0, h1, g0, g1, d0, d1,
             gsem0, gsem1, wsem0, wsem1, hsem0, hsem1):
    wid = lax.axis_index("s") * NUM_CORES + lax.axis_index("c")
    base = wid * BATCHES_PER_W

    pltpu.sync_copy(xp_hbm.at[pl.ds(base * L, BATCHES_PER_W * L)], idx_v)
    pltpu.sync_copy(post_hbm, post_v)

    gaths = (g0, g1)
    dests = (d0, d1)
    halvs = (h0, h1)
    gsems = (gsem0, gsem1)
    wsems = (wsem0, wsem1)
    hsems = (hsem0, hsem1)
    gathers = {}
    writes = {}

    def issue_chunk(c, s):
        ds = [
            pltpu.async_copy(
                pairs_hbm.at[idx_v.at[pl.ds(c * L, 128)]],
                gaths[s].at[pl.ds(0, 128)], gsems[s]),
            pltpu.async_copy(
                pairs_hbm.at[idx_v.at[pl.ds(c * L + 128, 72)]],
                gaths[s].at[pl.ds(128, 72)], gsems[s]),
            pltpu.async_copy(
                xh_hbm.at[pl.ds((base + c) * LP, LP)],
                halvs[s], hsems[s]),
        ]
        return ds

    def compute_chunk(s):
        gath, dest, hv = gaths[s], dests[s], halvs[s]

        def group(g, carry):
            rows = g * 16 + lax.iota(jnp.int32, 16)
            half = hv[pl.ds(g * 16, 16)]
            cols0 = half * DIM

            def col(d, carry2):
                dvec = jnp.full((16,), d, jnp.int32)
                vals = plsc.load_gather(gath, [rows, cols0 + dvec])
                p = post_v[pl.ds(d * LP + g * 16, 16)]
                plsc.store_scatter(dest, [rows, dvec], vals + p)
                return carry2

            return lax.fori_loop(0, DIM, col, carry, unroll=8)

        lax.fori_loop(0, GROUPS, group, 0)

    def issue_write(c, s):
        return pltpu.async_copy(
            dests[s].at[pl.ds(0, L)], out_hbm.at[base + c], wsems[s])

    for c in range(BATCHES_PER_W):
        s = c % 2
        if c >= 2:
            writes[c - 2].wait()
        gathers[c] = issue_chunk(c, s)
        if c >= 1:
            for d in gathers[c - 1]:
                d.wait()
            compute_chunk(1 - s)
            writes[c - 1] = issue_write(c - 1, 1 - s)

    last = BATCHES_PER_W - 1
    for d in gathers[last]:
        d.wait()
    compute_chunk(last % 2)
    writes[last] = issue_write(last, last % 2)
    writes[last - 1].wait()
    writes[last].wait()


@jax.jit
def _embed(xp, xh, w_word, pos_t):
    mesh = plsc.VectorSubcoreMesh(core_axis_name="c", subcore_axis_name="s")
    run = pl.kernel(
        _sc_body,
        out_type=jax.ShapeDtypeStruct((B, L, DIM), jnp.float32),
        mesh=mesh,
        scratch_types=[
            pltpu.VMEM((BATCHES_PER_W * L,), jnp.int32),
            pltpu.VMEM((DIM * LP,), jnp.float32),
            pltpu.VMEM((LP,), jnp.int32),
            pltpu.VMEM((LP,), jnp.int32),
            pltpu.VMEM((LP, 2 * DIM), jnp.float32),
            pltpu.VMEM((LP, 2 * DIM), jnp.float32),
            pltpu.VMEM((LP, DIM), jnp.float32),
            pltpu.VMEM((LP, DIM), jnp.float32),
            pltpu.SemaphoreType.DMA,
            pltpu.SemaphoreType.DMA,
            pltpu.SemaphoreType.DMA,
            pltpu.SemaphoreType.DMA,
            pltpu.SemaphoreType.DMA,
            pltpu.SemaphoreType.DMA,
        ],
        compiler_params=pltpu.CompilerParams(needs_layout_passes=False),
    )
    return run(xp, xh, w_word, pos_t)


def kernel(x, W_word, W_pos):
    xi = x.astype(jnp.int32)
    xp = (xi >> 1).reshape(B * L)
    xh = jnp.pad(xi & 1, ((0, 0), (0, LP - L))).reshape(B * LP)
    pairs = W_word.reshape(VOCAB // 2, 2 * DIM)
    pos_t = jnp.pad(W_pos.astype(jnp.float32).T,
                    ((0, 0), (0, LP - L))).reshape(DIM * LP)
    return _embed(xp, xh, pairs, pos_t)


# speedup vs baseline: 1.3484x; 1.3484x over previous
import jax
import jax.numpy as jnp
from jax import lax
from jax.experimental import pallas as pl
from jax.experimental.pallas import tpu as pltpu
from jax.experimental.pallas import tpu_sc as plsc

VOCAB = 1000000
CTX = 200
DIM = 64
B = 1024
L = 200

NUM_CORES = 2
NUM_SUBCORES = 16
NW = NUM_CORES * NUM_SUBCORES
LG = 8
BG = NW // LG
LPW = L // LG
BPW = B // BG
CPW = LPW


def _sc_body(xt_hbm, wpad_hbm, post_hbm, out_hbm,
             idx_v, post_v, g0, g1, d0, d1,
             gsem0, gsem1, wsem0, wsem1):
    wid = lax.axis_index("s") * NUM_CORES + lax.axis_index("c")
    li = wid // BG
    bi = wid - li * BG

    pltpu.sync_copy(xt_hbm.at[pl.ds(wid * LPW * BPW, LPW * BPW)], idx_v)
    pltpu.sync_copy(post_hbm, post_v)

    gaths = (g0, g1)
    dests = (d0, d1)
    gsems = (gsem0, gsem1)
    wsems = (wsem0, wsem1)
    gathers = {}
    writes = {}

    def issue_gathers(c, s):
        return [
            pltpu.async_copy(
                wpad_hbm.at[idx_v.at[pl.ds(c * BPW, 128)]],
                gaths[s].at[pl.ds(0, 128)], gsems[s]),
            pltpu.async_copy(
                wpad_hbm.at[idx_v.at[pl.ds(c * BPW + 128, 128)]],
                gaths[s].at[pl.ds(128, 128)], gsems[s]),
        ]

    def compute_chunk(c, s):
        gath, dest = gaths[s], dests[s]
        labs = li * LPW + c

        def bgroup(g, carry):
            rows = g * 16 + lax.iota(jnp.int32, 16)
            lvec = jnp.full((16,), labs, jnp.int32)

            def col(d, carry2):
                dvec = jnp.full((16,), d, jnp.int32)
                vals = plsc.load_gather(gath, [rows, dvec])
                p = plsc.load_gather(post_v, [dvec, lvec])
                dest[d, pl.ds(g * 16, 16)] = vals + p
                return carry2

            return lax.fori_loop(0, DIM, col, carry, unroll=8)

        lax.fori_loop(0, BPW // 16, bgroup, 0)

    def issue_write(c, s):
        return pltpu.async_copy(
            dests[s],
            out_hbm.at[pl.ds((li * LPW + c) * DIM, DIM), pl.ds(bi * BPW, BPW)],
            wsems[s])

    for c in range(CPW):
        s = c % 2
        if c >= 2:
            writes[c - 2].wait()
        gathers[c] = issue_gathers(c, s)
        if c >= 1:
            for d in gathers[c - 1]:
                d.wait()
            compute_chunk(c - 1, 1 - s)
            writes[c - 1] = issue_write(c - 1, 1 - s)

    last = CPW - 1
    for d in gathers[last]:
        d.wait()
    compute_chunk(last, last % 2)
    writes[last] = issue_write(last, last % 2)
    writes[last - 1].wait()
    writes[last].wait()


@jax.jit
def _embed(xt, wpad, pos_t):
    mesh = plsc.VectorSubcoreMesh(core_axis_name="c", subcore_axis_name="s")
    run = pl.kernel(
        _sc_body,
        out_type=jax.ShapeDtypeStruct((L * DIM, B), jnp.float32),
        mesh=mesh,
        scratch_types=[
            pltpu.VMEM((LPW * BPW,), jnp.int32),
            pltpu.VMEM((DIM, L), jnp.float32),
            pltpu.VMEM((BPW, 2 * DIM), jnp.float32),
            pltpu.VMEM((BPW, 2 * DIM), jnp.float32),
            pltpu.VMEM((DIM, BPW), jnp.float32),
            pltpu.VMEM((DIM, BPW), jnp.float32),
            pltpu.SemaphoreType.DMA,
            pltpu.SemaphoreType.DMA,
            pltpu.SemaphoreType.DMA,
            pltpu.SemaphoreType.DMA,
        ],
        compiler_params=pltpu.CompilerParams(needs_layout_passes=False),
    )
    return run(xt, wpad, pos_t)


def kernel(x, W_word, W_pos):
    xt = (x.astype(jnp.int32).T
          .reshape(LG, LPW, BG, BPW).transpose(0, 2, 1, 3).reshape(-1))
    wpad = jnp.pad(W_word, ((0, 0), (0, DIM)))
    pos_t = W_pos.astype(jnp.float32).T
    out_t = _embed(xt, wpad, pos_t)
    return out_t.reshape(L, DIM, B).transpose(2, 0, 1)
